# baseline (speedup 1.0000x reference)
import functools
import math

import jax
import jax.numpy as jnp
from jax import lax
from jax.experimental import pallas as pl
from jax.experimental.pallas import tpu as pltpu
from jax.experimental.pallas import tpu_sc as plsc

N = 16384
RATIO = 0.1
K = 5
M = math.ceil(RATIO * N)
ROWS = 128
COLS = N // ROWS
CBLK = 128
NBLK = (M + CBLK - 1) // CBLK
MPAD = NBLK * CBLK
GTOT = M * K
GCHUNK = 128
NCH = (GTOT + GCHUNK - 1) // GCHUNK
GPAD = NCH * GCHUNK
NWORK = 32


def _argmax_first(va, ia):
    size = va.shape[0]
    while size > 1:
        h = size // 2
        va1, va2 = va[:h], va[h:]
        ia1, ia2 = ia[:h], ia[h:]
        win = (va1 > va2) | ((va1 == va2) & (ia1 < ia2))
        va = jnp.where(win, va1, va2)
        ia = jnp.where(win, ia1, ia2)
        size = h
    w = va.shape[1]
    while w > 1:
        h = w // 2
        va1, va2 = va[:, :h], va[:, h:w]
        ia1, ia2 = ia[:, :h], ia[:, h:w]
        win = (va1 > va2) | ((va1 == va2) & (ia1 < ia2))
        va = jnp.where(win, va1, va2)
        ia = jnp.where(win, ia1, ia2)
        w = h
    return va[0, 0], ia[0, 0]


def _fps_body(px_ref, py_ref, pz_ref, idx_out, cx_out, cy_out, cz_out, d_ref):
    lin = (lax.broadcasted_iota(jnp.int32, (ROWS, COLS), 0) * COLS
           + lax.broadcasted_iota(jnp.int32, (ROWS, COLS), 1))
    lane = lax.broadcasted_iota(jnp.int32, (1, COLS), 1)

    px = px_ref[...]
    py = py_ref[...]
    pz = pz_ref[...]
    cx0 = px[0, 0]
    cy0 = py[0, 0]
    cz0 = pz[0, 0]
    idx_out[0] = jnp.int32(0)
    cx_out[0] = cx0
    cy_out[0] = cy0
    cz_out[0] = cz0
    dx = px - cx0
    dy = py - cy0
    dz = pz - cz0
    d_ref[...] = dx * dx + dy * dy + dz * dz

    def body(i, carry):
        d = d_ref[...]
        _, nxt = _argmax_first(d, lin)
        idx_out[i] = nxt
        r = nxt >> 7
        c = nxt & 127
        lm = lane == c
        cx = jnp.sum(jnp.where(lm, px_ref[pl.ds(r, 1), :], 0.0))
        cy = jnp.sum(jnp.where(lm, py_ref[pl.ds(r, 1), :], 0.0))
        cz = jnp.sum(jnp.where(lm, pz_ref[pl.ds(r, 1), :], 0.0))
        cx_out[i] = cx
        cy_out[i] = cy
        cz_out[i] = cz
        ddx = px_ref[...] - cx
        ddy = py_ref[...] - cy
        ddz = pz_ref[...] - cz
        dd = ddx * ddx + ddy * ddy + ddz * ddz
        d_ref[...] = jnp.minimum(d, dd)
        return carry

    lax.fori_loop(1, M, body, jnp.int32(0))


def _fps_call(pxm, pym, pzm):
    out_shape = [
        jax.ShapeDtypeStruct((M,), jnp.int32),
        jax.ShapeDtypeStruct((M,), jnp.float32),
        jax.ShapeDtypeStruct((M,), jnp.float32),
        jax.ShapeDtypeStruct((M,), jnp.float32),
    ]
    return pl.pallas_call(
        _fps_body,
        out_shape=out_shape,
        out_specs=[pl.BlockSpec(memory_space=pltpu.SMEM)] * 4,
        scratch_shapes=[pltpu.VMEM((ROWS, COLS), jnp.float32)],
    )(pxm, pym, pzm)


def _knn_body(cx_ref, cy_ref, cz_ref, px_ref, py_ref, pz_ref, out_ref, d2_ref):
    cx = jnp.reshape(cx_ref[...], (CBLK, 1))
    cy = jnp.reshape(cy_ref[...], (CBLK, 1))
    cz = jnp.reshape(cz_ref[...], (CBLK, 1))
    px = px_ref[...]
    py = py_ref[...]
    pz = pz_ref[...]
    dx = cx - px
    dy = cy - py
    dz = cz - pz
    d2_ref[...] = dx * dx + dy * dy + dz * dz

    iota = lax.broadcasted_iota(jnp.int32, (CBLK, N), 1)
    li = lax.broadcasted_iota(jnp.int32, (CBLK, 8), 1)
    acc = jnp.zeros((CBLK, 8), jnp.int32)
    for k in range(K):
        d2 = d2_ref[...]
        mv = jnp.min(d2, axis=1, keepdims=True)
        cand = jnp.where(d2 == mv, iota, jnp.int32(N))
        ik = jnp.min(cand, axis=1, keepdims=True)
        acc = jnp.where(li == k, ik, acc)
        d2_ref[...] = jnp.where(iota == ik, jnp.float32(jnp.inf), d2)
    out_ref[0] = acc


def _knn_call(cxp, cyp, czp, px1, py1, pz1):
    grid = (NBLK,)
    cen_spec = pl.BlockSpec((1, 1, CBLK), lambda b: (b, 0, 0))
    pts_spec = pl.BlockSpec((1, N), lambda b: (0, 0))
    return pl.pallas_call(
        _knn_body,
        grid=grid,
        in_specs=[cen_spec, cen_spec, cen_spec, pts_spec, pts_spec, pts_spec],
        out_specs=pl.BlockSpec((1, CBLK, 8), lambda b: (b, 0, 0)),
        out_shape=jax.ShapeDtypeStruct((NBLK, CBLK, 8), jnp.int32),
        scratch_shapes=[pltpu.VMEM((CBLK, N), jnp.float32)],
    )(cxp, cyp, czp, px1, py1, pz1)


def _gather_body(idx_hbm, tx_hbm, ty_hbm, tz_hbm,
                 gx_hbm, gy_hbm, gz_hbm, idx_v, row_v, sem):
    wid = lax.axis_index("s") * 2 + lax.axis_index("c")

    def do_chunk(c):
        base = c * GCHUNK
        pltpu.sync_copy(idx_hbm.at[pl.ds(base, GCHUNK)], idx_v)
        for t_hbm, g_hbm in ((tx_hbm, gx_hbm), (ty_hbm, gy_hbm),
                             (tz_hbm, gz_hbm)):
            pltpu.async_copy(t_hbm.at[idx_v], row_v, sem).wait()
            pltpu.sync_copy(row_v, g_hbm.at[pl.ds(base, GCHUNK)])

    for r in range((NCH + NWORK - 1) // NWORK):
        c = wid + r * NWORK

        @pl.when(c < NCH)
        def _():
            do_chunk(c)


def _gather_call(idx_pad, px, py, pz):
    mesh = plsc.VectorSubcoreMesh(core_axis_name="c", subcore_axis_name="s")
    f = pl.kernel(
        _gather_body,
        out_type=[jax.ShapeDtypeStruct((GPAD,), jnp.float32)] * 3,
        mesh=mesh,
        scratch_types=[
            pltpu.VMEM((GCHUNK,), jnp.int32),
            pltpu.VMEM((GCHUNK,), jnp.float32),
            pltpu.SemaphoreType.DMA,
        ],
    )
    return f(idx_pad, px, py, pz)


def kernel(x, pos, batch):
    px = pos[:, 0]
    py = pos[:, 1]
    pz = pos[:, 2]
    pxm = px.reshape(ROWS, COLS)
    pym = py.reshape(ROWS, COLS)
    pzm = pz.reshape(ROWS, COLS)

    fps_idx, cx, cy, cz = _fps_call(pxm, pym, pzm)
    centroids = jnp.stack([cx, cy, cz], axis=1)

    pad = MPAD - M
    cxp = jnp.concatenate([cx, jnp.zeros((pad,), jnp.float32)]).reshape(NBLK, 1, CBLK)
    cyp = jnp.concatenate([cy, jnp.zeros((pad,), jnp.float32)]).reshape(NBLK, 1, CBLK)
    czp = jnp.concatenate([cz, jnp.zeros((pad,), jnp.float32)]).reshape(NBLK, 1, CBLK)

    nbr8 = _knn_call(cxp, cyp, czp,
                     px.reshape(1, N), py.reshape(1, N), pz.reshape(1, N))
    nbr = nbr8[:, :, :K].reshape(MPAD * K)[: GTOT]

    idx_pad = jnp.concatenate([nbr, jnp.zeros((GPAD - GTOT,), jnp.int32)])
    gx, gy, gz = _gather_call(idx_pad, px, py, pz)
    groups = jnp.stack([gx[:GTOT], gy[:GTOT], gz[:GTOT]], axis=1)
    return centroids, groups


# speedup vs baseline: 1.3867x; 1.3867x over previous
import functools
import math

import jax
import jax.numpy as jnp
from jax import lax
from jax.experimental import pallas as pl
from jax.experimental.pallas import tpu as pltpu
from jax.experimental.pallas import tpu_sc as plsc

N = 16384
RATIO = 0.1
K = 5
M = math.ceil(RATIO * N)
ROWS = 128
COLS = N // ROWS
CBLK = 128
NBLK = (M + CBLK - 1) // CBLK
MPAD = NBLK * CBLK
GTOT = M * K
GCHUNK = 128
NCH = (GTOT + GCHUNK - 1) // GCHUNK
GPAD = NCH * GCHUNK
NWORK = 32


def _fps_body(px_ref, py_ref, pz_ref, idx_out, cx_out, cy_out, cz_out, d_ref):
    lane = lax.broadcasted_iota(jnp.int32, (1, COLS), 1)

    px = px_ref[...]
    py = py_ref[...]
    pz = pz_ref[...]
    cx0 = px[0, 0]
    cy0 = py[0, 0]
    cz0 = pz[0, 0]
    idx_out[0] = jnp.int32(0)
    cx_out[0] = cx0
    cy_out[0] = cy0
    cz_out[0] = cz0
    dx = px - cx0
    dy = py - cy0
    dz = pz - cz0
    d_ref[...] = dx * dx + dy * dy + dz * dz

    def body(i, carry):
        lin = (lax.broadcasted_iota(jnp.int32, (ROWS, COLS), 0) * COLS
               + lax.broadcasted_iota(jnp.int32, (ROWS, COLS), 1))
        d = d_ref[...]
        mx = jnp.max(d)
        nxt = jnp.min(jnp.where(d == mx, lin, jnp.int32(N)))
        idx_out[i] = nxt
        r = nxt >> 7
        c = nxt & 127
        lm = lane == c
        cx = jnp.sum(jnp.where(lm, px_ref[pl.ds(r, 1), :], 0.0))
        cy = jnp.sum(jnp.where(lm, py_ref[pl.ds(r, 1), :], 0.0))
        cz = jnp.sum(jnp.where(lm, pz_ref[pl.ds(r, 1), :], 0.0))
        cx_out[i] = cx
        cy_out[i] = cy
        cz_out[i] = cz
        ddx = px_ref[...] - cx
        ddy = py_ref[...] - cy
        ddz = pz_ref[...] - cz
        dd = ddx * ddx + ddy * ddy + ddz * ddz
        d_ref[...] = jnp.minimum(d, dd)
        return carry

    lax.fori_loop(1, M, body, jnp.int32(0))


def _fps_call(pxm, pym, pzm):
    out_shape = [
        jax.ShapeDtypeStruct((M,), jnp.int32),
        jax.ShapeDtypeStruct((M,), jnp.float32),
        jax.ShapeDtypeStruct((M,), jnp.float32),
        jax.ShapeDtypeStruct((M,), jnp.float32),
    ]
    return pl.pallas_call(
        _fps_body,
        out_shape=out_shape,
        out_specs=[pl.BlockSpec(memory_space=pltpu.SMEM)] * 4,
        scratch_shapes=[pltpu.VMEM((ROWS, COLS), jnp.float32)],
    )(pxm, pym, pzm)


def _knn_body(cx_ref, cy_ref, cz_ref, px_ref, py_ref, pz_ref, out_ref, d2_ref):
    cx = jnp.reshape(cx_ref[...], (CBLK, 1))
    cy = jnp.reshape(cy_ref[...], (CBLK, 1))
    cz = jnp.reshape(cz_ref[...], (CBLK, 1))
    px = px_ref[...]
    py = py_ref[...]
    pz = pz_ref[...]
    dx = cx - px
    dy = cy - py
    dz = cz - pz
    d2_ref[...] = dx * dx + dy * dy + dz * dz

    iota = lax.broadcasted_iota(jnp.int32, (CBLK, N), 1)
    li = lax.broadcasted_iota(jnp.int32, (CBLK, 8), 1)
    acc = jnp.zeros((CBLK, 8), jnp.int32)
    for k in range(K):
        d2 = d2_ref[...]
        mv = jnp.min(d2, axis=1, keepdims=True)
        cand = jnp.where(d2 == mv, iota, jnp.int32(N))
        ik = jnp.min(cand, axis=1, keepdims=True)
        acc = jnp.where(li == k, ik, acc)
        d2_ref[...] = jnp.where(iota == ik, jnp.float32(jnp.inf), d2)
    out_ref[0] = acc


def _knn_call(cxp, cyp, czp, px1, py1, pz1):
    grid = (NBLK,)
    cen_spec = pl.BlockSpec((1, 1, CBLK), lambda b: (b, 0, 0))
    pts_spec = pl.BlockSpec((1, N), lambda b: (0, 0))
    return pl.pallas_call(
        _knn_body,
        grid=grid,
        in_specs=[cen_spec, cen_spec, cen_spec, pts_spec, pts_spec, pts_spec],
        out_specs=pl.BlockSpec((1, CBLK, 8), lambda b: (b, 0, 0)),
        out_shape=jax.ShapeDtypeStruct((NBLK, CBLK, 8), jnp.int32),
        scratch_shapes=[pltpu.VMEM((CBLK, N), jnp.float32)],
    )(cxp, cyp, czp, px1, py1, pz1)


def _gather_body(idx_hbm, tx_hbm, ty_hbm, tz_hbm,
                 gx_hbm, gy_hbm, gz_hbm, idx_v, row_v, sem):
    wid = lax.axis_index("s") * 2 + lax.axis_index("c")

    def do_chunk(c):
        base = c * GCHUNK
        pltpu.sync_copy(idx_hbm.at[pl.ds(base, GCHUNK)], idx_v)
        for t_hbm, g_hbm in ((tx_hbm, gx_hbm), (ty_hbm, gy_hbm),
                             (tz_hbm, gz_hbm)):
            pltpu.async_copy(t_hbm.at[idx_v], row_v, sem).wait()
            pltpu.sync_copy(row_v, g_hbm.at[pl.ds(base, GCHUNK)])

    for r in range((NCH + NWORK - 1) // NWORK):
        c = wid + r * NWORK

        @pl.when(c < NCH)
        def _():
            do_chunk(c)


def _gather_call(idx_pad, px, py, pz):
    mesh = plsc.VectorSubcoreMesh(core_axis_name="c", subcore_axis_name="s")
    f = pl.kernel(
        _gather_body,
        out_type=[jax.ShapeDtypeStruct((GPAD,), jnp.float32)] * 3,
        mesh=mesh,
        scratch_types=[
            pltpu.VMEM((GCHUNK,), jnp.int32),
            pltpu.VMEM((GCHUNK,), jnp.float32),
            pltpu.SemaphoreType.DMA,
        ],
    )
    return f(idx_pad, px, py, pz)


def kernel(x, pos, batch):
    px = pos[:, 0]
    py = pos[:, 1]
    pz = pos[:, 2]
    pxm = px.reshape(ROWS, COLS)
    pym = py.reshape(ROWS, COLS)
    pzm = pz.reshape(ROWS, COLS)

    fps_idx, cx, cy, cz = _fps_call(pxm, pym, pzm)
    centroids = jnp.stack([cx, cy, cz], axis=1)

    pad = MPAD - M
    cxp = jnp.concatenate([cx, jnp.zeros((pad,), jnp.float32)]).reshape(NBLK, 1, CBLK)
    cyp = jnp.concatenate([cy, jnp.zeros((pad,), jnp.float32)]).reshape(NBLK, 1, CBLK)
    czp = jnp.concatenate([cz, jnp.zeros((pad,), jnp.float32)]).reshape(NBLK, 1, CBLK)

    nbr8 = _knn_call(cxp, cyp, czp,
                     px.reshape(1, N), py.reshape(1, N), pz.reshape(1, N))
    nbr = nbr8[:, :, :K].reshape(MPAD * K)[: GTOT]

    idx_pad = jnp.concatenate([nbr, jnp.zeros((GPAD - GTOT,), jnp.int32)])
    gx, gy, gz = _gather_call(idx_pad, px, py, pz)
    groups = jnp.stack([gx[:GTOT], gy[:GTOT], gz[:GTOT]], axis=1)
    return centroids, groups


# speedup vs baseline: 2.1225x; 1.5306x over previous
import functools
import math

import jax
import jax.numpy as jnp
from jax import lax
from jax.experimental import pallas as pl
from jax.experimental.pallas import tpu as pltpu
from jax.experimental.pallas import tpu_sc as plsc

N = 16384
RATIO = 0.1
K = 5
M = math.ceil(RATIO * N)
ROWS = 128
COLS = N // ROWS
CBLK = 128
NBLK = (M + CBLK - 1) // CBLK
MPAD = NBLK * CBLK
GTOT = M * K
GCHUNK = 128
NCH = (GTOT + GCHUNK - 1) // GCHUNK
GPAD = NCH * GCHUNK
NWORK = 32


def _fps_body(px_ref, py_ref, pz_ref, pxs_ref, pys_ref, pzs_ref,
              idx_out, cx_out, cy_out, cz_out, d_ref):
    px = px_ref[...]
    py = py_ref[...]
    pz = pz_ref[...]
    cx0 = px[0, 0]
    cy0 = py[0, 0]
    cz0 = pz[0, 0]
    idx_out[0] = jnp.int32(0)
    cx_out[0] = cx0
    cy_out[0] = cy0
    cz_out[0] = cz0
    dx = px - cx0
    dy = py - cy0
    dz = pz - cz0
    d_ref[...] = dx * dx + dy * dy + dz * dz

    def body(i, carry):
        linf = (lax.broadcasted_iota(jnp.int32, (ROWS, COLS), 0) * COLS
                + lax.broadcasted_iota(jnp.int32, (ROWS, COLS), 1)
                ).astype(jnp.float32)
        va = d_ref[...]
        ia = linf
        size = ROWS
        while size > 1:
            h = size // 2
            va1, va2 = va[:h], va[h:]
            ia1, ia2 = ia[:h], ia[h:]
            win = (va1 > va2) | ((va1 == va2) & (ia1 < ia2))
            va = jnp.where(win, va1, va2)
            ia = jnp.where(win, ia1, ia2)
            size = h
        mxk = jnp.max(va, axis=1, keepdims=True)
        cand = jnp.where(va == mxk, ia, jnp.float32(3.0e7))
        nxt = jnp.min(cand).astype(jnp.int32)
        idx_out[i] = nxt
        cx = pxs_ref[nxt]
        cy = pys_ref[nxt]
        cz = pzs_ref[nxt]
        cx_out[i] = cx
        cy_out[i] = cy
        cz_out[i] = cz
        ddx = px_ref[...] - cx
        ddy = py_ref[...] - cy
        ddz = pz_ref[...] - cz
        dd = ddx * ddx + ddy * ddy + ddz * ddz
        d_ref[...] = jnp.minimum(d_ref[...], dd)
        return carry

    lax.fori_loop(1, M, body, jnp.int32(0))


def _fps_call(pxm, pym, pzm, pxs, pys, pzs):
    out_shape = [
        jax.ShapeDtypeStruct((M,), jnp.int32),
        jax.ShapeDtypeStruct((M,), jnp.float32),
        jax.ShapeDtypeStruct((M,), jnp.float32),
        jax.ShapeDtypeStruct((M,), jnp.float32),
    ]
    return pl.pallas_call(
        _fps_body,
        out_shape=out_shape,
        in_specs=[
            pl.BlockSpec((ROWS, COLS), lambda: (0, 0)),
            pl.BlockSpec((ROWS, COLS), lambda: (0, 0)),
            pl.BlockSpec((ROWS, COLS), lambda: (0, 0)),
            pl.BlockSpec(memory_space=pltpu.SMEM),
            pl.BlockSpec(memory_space=pltpu.SMEM),
            pl.BlockSpec(memory_space=pltpu.SMEM),
        ],
        out_specs=[pl.BlockSpec(memory_space=pltpu.SMEM)] * 4,
        scratch_shapes=[pltpu.VMEM((ROWS, COLS), jnp.float32)],
    )(pxm, pym, pzm, pxs, pys, pzs)


def _knn_body(cx_ref, cy_ref, cz_ref, px_ref, py_ref, pz_ref, out_ref, d2_ref):
    cx = jnp.reshape(cx_ref[...], (CBLK, 1))
    cy = jnp.reshape(cy_ref[...], (CBLK, 1))
    cz = jnp.reshape(cz_ref[...], (CBLK, 1))
    px = px_ref[...]
    py = py_ref[...]
    pz = pz_ref[...]
    dx = cx - px
    dy = cy - py
    dz = cz - pz
    d2_ref[...] = dx * dx + dy * dy + dz * dz

    iota = lax.broadcasted_iota(jnp.int32, (CBLK, N), 1)
    li = lax.broadcasted_iota(jnp.int32, (CBLK, 8), 1)
    acc = jnp.zeros((CBLK, 8), jnp.int32)
    for k in range(K):
        d2 = d2_ref[...]
        mv = jnp.min(d2, axis=1, keepdims=True)
        cand = jnp.where(d2 == mv, iota, jnp.int32(N))
        ik = jnp.min(cand, axis=1, keepdims=True)
        acc = jnp.where(li == k, ik, acc)
        d2_ref[...] = jnp.where(iota == ik, jnp.float32(jnp.inf), d2)
    out_ref[0] = acc


def _knn_call(cxp, cyp, czp, px1, py1, pz1):
    grid = (NBLK,)
    cen_spec = pl.BlockSpec((1, 1, CBLK), lambda b: (b, 0, 0))
    pts_spec = pl.BlockSpec((1, N), lambda b: (0, 0))
    return pl.pallas_call(
        _knn_body,
        grid=grid,
        in_specs=[cen_spec, cen_spec, cen_spec, pts_spec, pts_spec, pts_spec],
        out_specs=pl.BlockSpec((1, CBLK, 8), lambda b: (b, 0, 0)),
        out_shape=jax.ShapeDtypeStruct((NBLK, CBLK, 8), jnp.int32),
        scratch_shapes=[pltpu.VMEM((CBLK, N), jnp.float32)],
    )(cxp, cyp, czp, px1, py1, pz1)


def _gather_body(idx_hbm, tx_hbm, ty_hbm, tz_hbm,
                 gx_hbm, gy_hbm, gz_hbm, idx_v, row_v, sem):
    wid = lax.axis_index("s") * 2 + lax.axis_index("c")

    def do_chunk(c):
        base = c * GCHUNK
        pltpu.sync_copy(idx_hbm.at[pl.ds(base, GCHUNK)], idx_v)
        for t_hbm, g_hbm in ((tx_hbm, gx_hbm), (ty_hbm, gy_hbm),
                             (tz_hbm, gz_hbm)):
            pltpu.async_copy(t_hbm.at[idx_v], row_v, sem).wait()
            pltpu.sync_copy(row_v, g_hbm.at[pl.ds(base, GCHUNK)])

    for r in range((NCH + NWORK - 1) // NWORK):
        c = wid + r * NWORK

        @pl.when(c < NCH)
        def _():
            do_chunk(c)


def _gather_call(idx_pad, px, py, pz):
    mesh = plsc.VectorSubcoreMesh(core_axis_name="c", subcore_axis_name="s")
    f = pl.kernel(
        _gather_body,
        out_type=[jax.ShapeDtypeStruct((GPAD,), jnp.float32)] * 3,
        mesh=mesh,
        scratch_types=[
            pltpu.VMEM((GCHUNK,), jnp.int32),
            pltpu.VMEM((GCHUNK,), jnp.float32),
            pltpu.SemaphoreType.DMA,
        ],
    )
    return f(idx_pad, px, py, pz)


def kernel(x, pos, batch):
    px = pos[:, 0]
    py = pos[:, 1]
    pz = pos[:, 2]
    pxm = px.reshape(ROWS, COLS)
    pym = py.reshape(ROWS, COLS)
    pzm = pz.reshape(ROWS, COLS)

    fps_idx, cx, cy, cz = _fps_call(pxm, pym, pzm, px, py, pz)
    centroids = jnp.stack([cx, cy, cz], axis=1)

    pad = MPAD - M
    cxp = jnp.concatenate([cx, jnp.zeros((pad,), jnp.float32)]).reshape(NBLK, 1, CBLK)
    cyp = jnp.concatenate([cy, jnp.zeros((pad,), jnp.float32)]).reshape(NBLK, 1, CBLK)
    czp = jnp.concatenate([cz, jnp.zeros((pad,), jnp.float32)]).reshape(NBLK, 1, CBLK)

    nbr8 = _knn_call(cxp, cyp, czp,
                     px.reshape(1, N), py.reshape(1, N), pz.reshape(1, N))
    nbr = nbr8[:, :, :K].reshape(MPAD * K)[: GTOT]

    idx_pad = jnp.concatenate([nbr, jnp.zeros((GPAD - GTOT,), jnp.int32)])
    gx, gy, gz = _gather_call(idx_pad, px, py, pz)
    groups = jnp.stack([gx[:GTOT], gy[:GTOT], gz[:GTOT]], axis=1)
    return centroids, groups


# speedup vs baseline: 2.2514x; 1.0607x over previous
import functools
import math

import jax
import jax.numpy as jnp
from jax import lax
from jax.experimental import pallas as pl
from jax.experimental.pallas import tpu as pltpu
from jax.experimental.pallas import tpu_sc as plsc

N = 16384
RATIO = 0.1
K = 5
M = math.ceil(RATIO * N)
ROWS = 128
COLS = N // ROWS
CBLK = 128
NBLK = (M + CBLK - 1) // CBLK
MPAD = NBLK * CBLK
GTOT = M * K
GCHUNK = 128
NCH = (GTOT + GCHUNK - 1) // GCHUNK
GPAD = NCH * GCHUNK
NWORK = 32


def _fps_body(px_ref, py_ref, pz_ref, pxs_ref, pys_ref, pzs_ref,
              idx_out, cx_out, cy_out, cz_out, lin_ref):
    px = px_ref[...]
    py = py_ref[...]
    pz = pz_ref[...]
    cx0 = px[0, 0]
    cy0 = py[0, 0]
    cz0 = pz[0, 0]
    idx_out[0] = jnp.int32(0)
    cx_out[0] = cx0
    cy_out[0] = cy0
    cz_out[0] = cz0
    dx = px - cx0
    dy = py - cy0
    dz = pz - cz0
    d0 = dx * dx + dy * dy + dz * dz
    lin_ref[...] = (lax.broadcasted_iota(jnp.int32, (ROWS, COLS), 0) * COLS
                    + lax.broadcasted_iota(jnp.int32, (ROWS, COLS), 1)
                    ).astype(jnp.float32)

    def body(i, d):
        va = d
        ia = lin_ref[...]
        size = ROWS
        while size > 1:
            h = size // 2
            va1, va2 = va[:h], va[h:]
            ia1, ia2 = ia[:h], ia[h:]
            win = (va1 > va2) | ((va1 == va2) & (ia1 < ia2))
            va = jnp.where(win, va1, va2)
            ia = jnp.where(win, ia1, ia2)
            size = h
        mxk = jnp.max(va, axis=1, keepdims=True)
        cand = jnp.where(va == mxk, ia, jnp.float32(3.0e7))
        nxt = jnp.min(cand).astype(jnp.int32)
        idx_out[i] = nxt
        cx = pxs_ref[nxt]
        cy = pys_ref[nxt]
        cz = pzs_ref[nxt]
        cx_out[i] = cx
        cy_out[i] = cy
        cz_out[i] = cz
        ddx = px_ref[...] - cx
        ddy = py_ref[...] - cy
        ddz = pz_ref[...] - cz
        dd = ddx * ddx + ddy * ddy + ddz * ddz
        return jnp.minimum(d, dd)

    lax.fori_loop(1, M, body, d0)


def _fps_call(pxm, pym, pzm, pxs, pys, pzs):
    out_shape = [
        jax.ShapeDtypeStruct((M,), jnp.int32),
        jax.ShapeDtypeStruct((M,), jnp.float32),
        jax.ShapeDtypeStruct((M,), jnp.float32),
        jax.ShapeDtypeStruct((M,), jnp.float32),
    ]
    return pl.pallas_call(
        _fps_body,
        out_shape=out_shape,
        in_specs=[
            pl.BlockSpec((ROWS, COLS), lambda: (0, 0)),
            pl.BlockSpec((ROWS, COLS), lambda: (0, 0)),
            pl.BlockSpec((ROWS, COLS), lambda: (0, 0)),
            pl.BlockSpec(memory_space=pltpu.SMEM),
            pl.BlockSpec(memory_space=pltpu.SMEM),
            pl.BlockSpec(memory_space=pltpu.SMEM),
        ],
        out_specs=[pl.BlockSpec(memory_space=pltpu.SMEM)] * 4,
        scratch_shapes=[pltpu.VMEM((ROWS, COLS), jnp.float32)],
    )(pxm, pym, pzm, pxs, pys, pzs)


def _knn_body(cx_ref, cy_ref, cz_ref, px_ref, py_ref, pz_ref, out_ref, d2_ref):
    cx = jnp.reshape(cx_ref[...], (CBLK, 1))
    cy = jnp.reshape(cy_ref[...], (CBLK, 1))
    cz = jnp.reshape(cz_ref[...], (CBLK, 1))
    px = px_ref[...]
    py = py_ref[...]
    pz = pz_ref[...]
    dx = cx - px
    dy = cy - py
    dz = cz - pz
    d2 = dx * dx + dy * dy + dz * dz
    d2_ref[...] = d2
    mv = jnp.min(d2, axis=1, keepdims=True)

    iotaf = lax.broadcasted_iota(jnp.int32, (CBLK, N), 1).astype(jnp.float32)
    li = lax.broadcasted_iota(jnp.int32, (CBLK, 8), 1)
    acc = jnp.zeros((CBLK, 8), jnp.int32)
    for k in range(K):
        d2 = d2_ref[...]
        cand = jnp.where(d2 == mv, iotaf, jnp.float32(3.0e7))
        ikf = jnp.min(cand, axis=1, keepdims=True)
        acc = jnp.where(li == k, ikf.astype(jnp.int32), acc)
        if k < K - 1:
            masked = jnp.where(iotaf == ikf, jnp.float32(jnp.inf), d2)
            d2_ref[...] = masked
            mv = jnp.min(masked, axis=1, keepdims=True)
    out_ref[0] = acc


def _knn_call(cxp, cyp, czp, px1, py1, pz1):
    grid = (NBLK,)
    cen_spec = pl.BlockSpec((1, 1, CBLK), lambda b: (b, 0, 0))
    pts_spec = pl.BlockSpec((1, N), lambda b: (0, 0))
    return pl.pallas_call(
        _knn_body,
        grid=grid,
        in_specs=[cen_spec, cen_spec, cen_spec, pts_spec, pts_spec, pts_spec],
        out_specs=pl.BlockSpec((1, CBLK, 8), lambda b: (b, 0, 0)),
        out_shape=jax.ShapeDtypeStruct((NBLK, CBLK, 8), jnp.int32),
        scratch_shapes=[pltpu.VMEM((CBLK, N), jnp.float32)],
    )(cxp, cyp, czp, px1, py1, pz1)


def _gather_body(idx_hbm, tx_hbm, ty_hbm, tz_hbm,
                 gx_hbm, gy_hbm, gz_hbm, idx_v, row_v, sem):
    wid = lax.axis_index("s") * 2 + lax.axis_index("c")

    def do_chunk(c):
        base = c * GCHUNK
        pltpu.sync_copy(idx_hbm.at[pl.ds(base, GCHUNK)], idx_v)
        for t_hbm, g_hbm in ((tx_hbm, gx_hbm), (ty_hbm, gy_hbm),
                             (tz_hbm, gz_hbm)):
            pltpu.async_copy(t_hbm.at[idx_v], row_v, sem).wait()
            pltpu.sync_copy(row_v, g_hbm.at[pl.ds(base, GCHUNK)])

    for r in range((NCH + NWORK - 1) // NWORK):
        c = wid + r * NWORK

        @pl.when(c < NCH)
        def _():
            do_chunk(c)


def _gather_call(idx_pad, px, py, pz):
    mesh = plsc.VectorSubcoreMesh(core_axis_name="c", subcore_axis_name="s")
    f = pl.kernel(
        _gather_body,
        out_type=[jax.ShapeDtypeStruct((GPAD,), jnp.float32)] * 3,
        mesh=mesh,
        scratch_types=[
            pltpu.VMEM((GCHUNK,), jnp.int32),
            pltpu.VMEM((GCHUNK,), jnp.float32),
            pltpu.SemaphoreType.DMA,
        ],
    )
    return f(idx_pad, px, py, pz)


def kernel(x, pos, batch):
    px = pos[:, 0]
    py = pos[:, 1]
    pz = pos[:, 2]
    pxm = px.reshape(ROWS, COLS)
    pym = py.reshape(ROWS, COLS)
    pzm = pz.reshape(ROWS, COLS)

    fps_idx, cx, cy, cz = _fps_call(pxm, pym, pzm, px, py, pz)
    centroids = jnp.stack([cx, cy, cz], axis=1)

    pad = MPAD - M
    cxp = jnp.concatenate([cx, jnp.zeros((pad,), jnp.float32)]).reshape(NBLK, 1, CBLK)
    cyp = jnp.concatenate([cy, jnp.zeros((pad,), jnp.float32)]).reshape(NBLK, 1, CBLK)
    czp = jnp.concatenate([cz, jnp.zeros((pad,), jnp.float32)]).reshape(NBLK, 1, CBLK)

    nbr8 = _knn_call(cxp, cyp, czp,
                     px.reshape(1, N), py.reshape(1, N), pz.reshape(1, N))
    nbr = nbr8[:, :, :K].reshape(MPAD * K)[: GTOT]

    idx_pad = jnp.concatenate([nbr, jnp.zeros((GPAD - GTOT,), jnp.int32)])
    gx, gy, gz = _gather_call(idx_pad, px, py, pz)
    groups = jnp.stack([gx[:GTOT], gy[:GTOT], gz[:GTOT]], axis=1)
    return centroids, groups
